# baseline (speedup 1.0000x reference)
import jax
import jax.numpy as jnp
from jax import lax
from jax.experimental import pallas as pl
from jax.experimental.pallas import tpu as pltpu
from jax.experimental.pallas import tpu_sc as plsc

N_NODES = 10000
D = 128
E = 320000
CHUNK = 128
NC = 2
NS = 16
NW = NC * NS
NJ = 80
E_PAD = NJ * NW * CHUNK
N_PAD = 10240
ROWS_PER_TILE = N_PAD // NS
ZROWS = 32


def _make_sc_agg():
    mesh = plsc.VectorSubcoreMesh(core_axis_name="c", subcore_axis_name="s")

    out_type = jax.ShapeDtypeStruct((NC, N_PAD, D), jnp.float32)
    scratch = [
        pltpu.VMEM((1, CHUNK), jnp.int32),
        pltpu.VMEM((1, CHUNK), jnp.int32),
        pltpu.VMEM((1, CHUNK), jnp.int32),
        pltpu.VMEM((1, CHUNK), jnp.int32),
        pltpu.VMEM((CHUNK, D), jnp.float32),
        pltpu.VMEM((CHUNK, D), jnp.float32),
        pltpu.VMEM((ZROWS, D), jnp.float32),
        pltpu.VMEM_SHARED((N_PAD, D), jnp.float32),
        pltpu.SemaphoreType.DMA,
    ]

    def body(feat, src, dst, out_acc, sidx0, sidx1, didx0, didx1,
             rows0, rows1, zbuf, acc_sh, gsem):
        cid = lax.axis_index("c")
        sid = lax.axis_index("s")
        wid = sid * NC + cid

        zv = jnp.zeros((16,), jnp.float32)

        def zfill(r, carry):
            for c in range(D // 16):
                zbuf[r, pl.ds(c * 16, 16)] = zv
            return carry

        lax.fori_loop(0, ZROWS, zfill, 0)
        r0 = sid * ROWS_PER_TILE

        def zero_body(t, carry):
            pltpu.sync_copy(zbuf, acc_sh.at[pl.ds(r0 + t * ZROWS, ZROWS)])
            return carry

        lax.fori_loop(0, ROWS_PER_TILE // ZROWS, zero_body, 0)
        plsc.subcore_barrier()

        def load_idx(c, sidx, didx):
            off = (c * NW + wid) * CHUNK
            pltpu.sync_copy(src.at[pl.ds(off, CHUNK)], sidx.at[0])
            pltpu.sync_copy(dst.at[pl.ds(off, CHUNK)], didx.at[0])

        def wait_gather():
            pltpu.make_async_copy(feat.at[sidx0.at[0]], rows0, gsem).wait()

        load_idx(0, sidx0, didx0)
        pltpu.async_copy(feat.at[sidx0.at[0]], rows0, gsem)

        def pair_body(i, carry):
            c0 = 2 * i
            load_idx(c0 + 1, sidx1, didx1)
            wait_gather()
            pltpu.async_copy(feat.at[sidx1.at[0]], rows1, gsem)
            pltpu.sync_copy(rows0, acc_sh.at[didx0.at[0]], add=True)

            @pl.when(c0 + 2 < NJ)
            def _():
                load_idx(c0 + 2, sidx0, didx0)
            wait_gather()

            @pl.when(c0 + 2 < NJ)
            def _():
                pltpu.async_copy(feat.at[sidx0.at[0]], rows0, gsem)
            pltpu.sync_copy(rows1, acc_sh.at[didx1.at[0]], add=True)
            return carry

        lax.fori_loop(0, NJ // 2, pair_body, 0)
        plsc.subcore_barrier()

        pltpu.sync_copy(acc_sh.at[pl.ds(r0, ROWS_PER_TILE)],
                        out_acc.at[cid, pl.ds(r0, ROWS_PER_TILE)])

    return pl.kernel(body, out_type=out_type, mesh=mesh,
                     scratch_types=scratch)


def _make_sc_deg():
    mesh = plsc.VectorSubcoreMesh(core_axis_name="c", subcore_axis_name="s")

    out_type = jax.ShapeDtypeStruct((NC, 8, N_PAD), jnp.float32)
    COLS = N_PAD // NS
    scratch = [
        pltpu.VMEM((NJ * CHUNK,), jnp.int32),
        pltpu.VMEM((N_PAD,), jnp.float32),
        pltpu.VMEM((NS * COLS,), jnp.float32),
        pltpu.VMEM((COLS,), jnp.float32),
        pltpu.VMEM_SHARED((NS * N_PAD,), jnp.float32),
    ]

    def body(dst, out_deg, slab_d, hist, red, obuf, deg_sh):
        cid = lax.axis_index("c")
        sid = lax.axis_index("s")
        wid = sid * NC + cid

        pltpu.sync_copy(dst.at[pl.ds(wid * NJ * CHUNK, NJ * CHUNK)], slab_d)

        zv = jnp.zeros((16,), jnp.float32)

        def zero_body(i, carry):
            hist[pl.ds(i * 16, 16)] = zv
            return carry

        lax.fori_loop(0, N_PAD // 16, zero_body, 0)

        onev = jnp.ones((16,), jnp.float32)

        def chunk_body(j, carry):
            for k in range(CHUNK // 16):
                idxv = slab_d[pl.ds(j * CHUNK + k * 16, 16)]
                plsc.addupdate_scatter(hist, [idxv], onev)
            return carry

        lax.fori_loop(0, NJ, chunk_body, 0)

        pltpu.sync_copy(hist, deg_sh.at[pl.ds(sid * N_PAD, N_PAD)])
        plsc.subcore_barrier()

        c0 = sid * COLS

        def pull_body(j, carry):
            pltpu.sync_copy(deg_sh.at[pl.ds(j * N_PAD + c0, COLS)],
                            red.at[pl.ds(j * COLS, COLS)])
            return carry

        lax.fori_loop(0, NS, pull_body, 0)

        def sum_body(t, carry):
            s = red[pl.ds(t * 16, 16)]
            for j in range(1, NS):
                s = s + red[pl.ds(j * COLS + t * 16, 16)]
            obuf[pl.ds(t * 16, 16)] = s
            return carry

        lax.fori_loop(0, COLS // 16, sum_body, 0)

        pltpu.sync_copy(obuf, out_deg.at[cid, 0, pl.ds(c0, COLS)])

    return pl.kernel(body, out_type=out_type, mesh=mesh,
                     scratch_types=scratch,
                     compiler_params=pltpu.CompilerParams(
                         needs_layout_passes=False))


_sc_agg = _make_sc_agg()
_sc_deg = _make_sc_deg()


def _combine(feat, p, inv, w_self, w_neigh, b, relu):
    R = 2000

    def body(feat_ref, p_ref, inv_ref, ws_ref, wn_ref, b_ref, out_ref):
        neigh = (p_ref[0] + p_ref[1]) * inv_ref[...]
        acc = jnp.dot(feat_ref[...], ws_ref[...],
                      preferred_element_type=jnp.float32)
        acc += jnp.dot(neigh, wn_ref[...], preferred_element_type=jnp.float32)
        acc += b_ref[...]
        if relu:
            acc = jnp.maximum(acc, 0.0)
        out_ref[...] = acc

    return pl.pallas_call(
        body,
        grid=(N_NODES // R,),
        in_specs=[
            pl.BlockSpec((R, D), lambda i: (i, 0)),
            pl.BlockSpec((NC, R, D), lambda i: (0, i, 0)),
            pl.BlockSpec((R, 1), lambda i: (i, 0)),
            pl.BlockSpec((D, D), lambda i: (0, 0)),
            pl.BlockSpec((D, D), lambda i: (0, 0)),
            pl.BlockSpec((1, D), lambda i: (0, 0)),
        ],
        out_specs=pl.BlockSpec((R, D), lambda i: (i, 0)),
        out_shape=jax.ShapeDtypeStruct((N_NODES, D), jnp.float32),
    )(feat, p, inv, w_self, w_neigh, b.reshape(1, D))


@jax.jit
def _impl(x, src, dst, W1_self, W1_neigh, b1, W2_self, W2_neigh, b2):
    accp1 = _sc_agg(x, src, dst)
    degp = _sc_deg(dst)
    deg = degp[0, 0, :] + degp[1, 0, :]
    inv = (1.0 / jnp.maximum(deg, 1.0)).reshape(N_PAD, 1)
    h = _combine(x, accp1, inv, W1_self, W1_neigh, b1, relu=True)
    accp2 = _sc_agg(h, src, dst)
    return _combine(h, accp2, inv, W2_self, W2_neigh, b2, relu=False)


def kernel(x, edge_index, W1_self, W1_neigh, b1, W2_self, W2_neigh, b2):
    src = edge_index[0].astype(jnp.int32)
    dst = edge_index[1].astype(jnp.int32)
    src_p = jnp.concatenate([src, jnp.zeros((E_PAD - E,), jnp.int32)])
    pad_dst = N_NODES + (jnp.arange(E_PAD - E, dtype=jnp.int32)
                         % (N_PAD - N_NODES))
    dst_p = jnp.concatenate([dst, pad_dst])
    return _impl(x, src_p, dst_p, W1_self, W1_neigh, b1,
                 W2_self, W2_neigh, b2)


# speedup vs baseline: 1.6898x; 1.6898x over previous
import jax
import jax.numpy as jnp
from jax import lax
from jax.experimental import pallas as pl
from jax.experimental.pallas import tpu as pltpu
from jax.experimental.pallas import tpu_sc as plsc

N_NODES = 10000
D = 128
E = 320000
CHUNK = 128
NC = 2
NS = 16
NW = NC * NS
N_CHUNKS = E // CHUNK
BASE_J = N_CHUNKS // NW
REM = N_CHUNKS - BASE_J * NW
E_W = E // NW
N_PAD = 10240
ROWS_PER_TILE = N_PAD // NS
ZROWS = 32


def _make_sc_agg():
    mesh = plsc.VectorSubcoreMesh(core_axis_name="c", subcore_axis_name="s")

    out_type = jax.ShapeDtypeStruct((NC, N_PAD, D), jnp.float32)
    scratch = [
        pltpu.VMEM((1, CHUNK), jnp.int32),
        pltpu.VMEM((1, CHUNK), jnp.int32),
        pltpu.VMEM((CHUNK, D), jnp.float32),
        pltpu.VMEM((ZROWS, D), jnp.float32),
        pltpu.VMEM_SHARED((N_PAD, D), jnp.float32),
    ]

    def body(feat, src, dst, out_acc, idx_s, idx_d, rows, zbuf, acc_sh):
        cid = lax.axis_index("c")
        sid = lax.axis_index("s")
        wid = sid * NC + cid

        zv = jnp.zeros((16,), jnp.float32)

        def zfill(r, carry):
            for c in range(D // 16):
                zbuf[r, pl.ds(c * 16, 16)] = zv
            return carry

        lax.fori_loop(0, ZROWS, zfill, 0)
        r0 = sid * ROWS_PER_TILE

        def zero_body(t, carry):
            pltpu.sync_copy(zbuf, acc_sh.at[pl.ds(r0 + t * ZROWS, ZROWS)])
            return carry

        lax.fori_loop(0, ROWS_PER_TILE // ZROWS, zero_body, 0)
        plsc.subcore_barrier()

        nj = BASE_J + jnp.where(wid < REM, 1, 0)

        def chunk_body(j, carry):
            off = (j * NW + wid) * CHUNK
            pltpu.sync_copy(src.at[pl.ds(off, CHUNK)], idx_s.at[0])
            pltpu.sync_copy(dst.at[pl.ds(off, CHUNK)], idx_d.at[0])
            pltpu.sync_copy(feat.at[idx_s.at[0]], rows)
            pltpu.sync_copy(rows, acc_sh.at[idx_d.at[0]], add=True)
            return carry

        lax.fori_loop(0, nj, chunk_body, 0)
        plsc.subcore_barrier()

        pltpu.sync_copy(acc_sh.at[pl.ds(r0, ROWS_PER_TILE)],
                        out_acc.at[cid, pl.ds(r0, ROWS_PER_TILE)])

    return pl.kernel(body, out_type=out_type, mesh=mesh,
                     scratch_types=scratch)


def _make_sc_deg():
    mesh = plsc.VectorSubcoreMesh(core_axis_name="c", subcore_axis_name="s")

    out_type = jax.ShapeDtypeStruct((NC, 8, N_PAD), jnp.float32)
    COLS = N_PAD // NS
    scratch = [
        pltpu.VMEM((E_W,), jnp.int32),
        pltpu.VMEM((N_PAD,), jnp.float32),
        pltpu.VMEM((NS * COLS,), jnp.float32),
        pltpu.VMEM((COLS,), jnp.float32),
        pltpu.VMEM_SHARED((NS * N_PAD,), jnp.float32),
    ]

    def body(dst, out_deg, slab_d, hist, red, obuf, deg_sh):
        cid = lax.axis_index("c")
        sid = lax.axis_index("s")
        wid = sid * NC + cid

        pltpu.sync_copy(dst.at[pl.ds(wid * E_W, E_W)], slab_d)

        zv = jnp.zeros((16,), jnp.float32)

        def zero_body(i, carry):
            hist[pl.ds(i * 16, 16)] = zv
            return carry

        lax.fori_loop(0, N_PAD // 16, zero_body, 0)

        onev = jnp.ones((16,), jnp.float32)

        def chunk_body(j, carry):
            for k in range(8):
                idxv = slab_d[pl.ds(j * CHUNK + k * 16, 16)]
                plsc.addupdate_scatter(hist, [idxv], onev)
            return carry

        lax.fori_loop(0, E_W // CHUNK, chunk_body, 0)

        pltpu.sync_copy(hist, deg_sh.at[pl.ds(sid * N_PAD, N_PAD)])
        plsc.subcore_barrier()

        c0 = sid * COLS

        def pull_body(j, carry):
            pltpu.sync_copy(deg_sh.at[pl.ds(j * N_PAD + c0, COLS)],
                            red.at[pl.ds(j * COLS, COLS)])
            return carry

        lax.fori_loop(0, NS, pull_body, 0)

        def sum_body(t, carry):
            s = red[pl.ds(t * 16, 16)]
            for j in range(1, NS):
                s = s + red[pl.ds(j * COLS + t * 16, 16)]
            obuf[pl.ds(t * 16, 16)] = s
            return carry

        lax.fori_loop(0, COLS // 16, sum_body, 0)

        pltpu.sync_copy(obuf, out_deg.at[cid, 0, pl.ds(c0, COLS)])

    return pl.kernel(body, out_type=out_type, mesh=mesh,
                     scratch_types=scratch,
                     compiler_params=pltpu.CompilerParams(
                         needs_layout_passes=False))


_sc_agg = _make_sc_agg()
_sc_deg = _make_sc_deg()


def _combine(feat, p, inv, w_self, w_neigh, b, relu):
    R = 2000

    def body(feat_ref, p_ref, inv_ref, ws_ref, wn_ref, b_ref, out_ref):
        neigh = (p_ref[0] + p_ref[1]) * inv_ref[...]
        acc = jnp.dot(feat_ref[...], ws_ref[...],
                      preferred_element_type=jnp.float32)
        acc += jnp.dot(neigh, wn_ref[...], preferred_element_type=jnp.float32)
        acc += b_ref[...]
        if relu:
            acc = jnp.maximum(acc, 0.0)
        out_ref[...] = acc

    return pl.pallas_call(
        body,
        grid=(N_NODES // R,),
        in_specs=[
            pl.BlockSpec((R, D), lambda i: (i, 0)),
            pl.BlockSpec((NC, R, D), lambda i: (0, i, 0)),
            pl.BlockSpec((R, 1), lambda i: (i, 0)),
            pl.BlockSpec((D, D), lambda i: (0, 0)),
            pl.BlockSpec((D, D), lambda i: (0, 0)),
            pl.BlockSpec((1, D), lambda i: (0, 0)),
        ],
        out_specs=pl.BlockSpec((R, D), lambda i: (i, 0)),
        out_shape=jax.ShapeDtypeStruct((N_NODES, D), jnp.float32),
    )(feat, p, inv, w_self, w_neigh, b.reshape(1, D))


@jax.jit
def _impl(x, src, dst, W1_self, W1_neigh, b1, W2_self, W2_neigh, b2):
    accp1 = _sc_agg(x, src, dst)
    degp = _sc_deg(dst)
    deg = degp[0, 0, :] + degp[1, 0, :]
    inv = (1.0 / jnp.maximum(deg, 1.0)).reshape(N_PAD, 1)
    h = _combine(x, accp1, inv, W1_self, W1_neigh, b1, relu=True)
    accp2 = _sc_agg(h, src, dst)
    return _combine(h, accp2, inv, W2_self, W2_neigh, b2, relu=False)


def kernel(x, edge_index, W1_self, W1_neigh, b1, W2_self, W2_neigh, b2):
    src = edge_index[0].astype(jnp.int32)
    dst = edge_index[1].astype(jnp.int32)
    return _impl(x, src, dst, W1_self, W1_neigh, b1,
                 W2_self, W2_neigh, b2)


# speedup vs baseline: 1.6903x; 1.0003x over previous
import jax
import jax.numpy as jnp
from jax import lax
from jax.experimental import pallas as pl
from jax.experimental.pallas import tpu as pltpu
from jax.experimental.pallas import tpu_sc as plsc

N_NODES = 10000
D = 128
E = 320000
CHUNK = 128
NC = 2
NS = 16
NW = NC * NS
N_CHUNKS = E // CHUNK
BASE_J = N_CHUNKS // NW
REM = N_CHUNKS - BASE_J * NW
E_W = E // NW
N_PAD = 10240
ROWS_PER_TILE = N_PAD // NS
ZROWS = 32


def _make_sc_agg():
    mesh = plsc.VectorSubcoreMesh(core_axis_name="c", subcore_axis_name="s")

    out_type = jax.ShapeDtypeStruct((NC, N_PAD, D), jnp.float32)
    scratch = [
        pltpu.VMEM((1, CHUNK), jnp.int32),
        pltpu.VMEM((1, CHUNK), jnp.int32),
        pltpu.VMEM((CHUNK, D), jnp.float32),
        pltpu.VMEM((ZROWS, D), jnp.float32),
        pltpu.VMEM_SHARED((N_PAD, D), jnp.float32),
    ]

    def body(feat, src, dst, out_acc, idx_s, idx_d, rows, zbuf, acc_sh):
        cid = lax.axis_index("c")
        sid = lax.axis_index("s")
        wid = sid * NC + cid

        zv = jnp.zeros((16,), jnp.float32)

        def zfill(r, carry):
            for c in range(D // 16):
                zbuf[r, pl.ds(c * 16, 16)] = zv
            return carry

        lax.fori_loop(0, ZROWS, zfill, 0)
        r0 = sid * ROWS_PER_TILE

        def zero_body(t, carry):
            pltpu.sync_copy(zbuf, acc_sh.at[pl.ds(r0 + t * ZROWS, ZROWS)])
            return carry

        lax.fori_loop(0, ROWS_PER_TILE // ZROWS, zero_body, 0)
        plsc.subcore_barrier()

        nj = BASE_J + jnp.where(wid < REM, 1, 0)

        def chunk_body(j, carry):
            off = (j * NW + wid) * CHUNK
            pltpu.sync_copy(src.at[pl.ds(off, CHUNK)], idx_s.at[0])
            pltpu.sync_copy(dst.at[pl.ds(off, CHUNK)], idx_d.at[0])
            pltpu.sync_copy(feat.at[idx_s.at[0]], rows)
            pltpu.sync_copy(rows, acc_sh.at[idx_d.at[0]], add=True)
            return carry

        lax.fori_loop(0, nj, chunk_body, 0)
        plsc.subcore_barrier()

        pltpu.sync_copy(acc_sh.at[pl.ds(r0, ROWS_PER_TILE)],
                        out_acc.at[cid, pl.ds(r0, ROWS_PER_TILE)])

    return pl.kernel(body, out_type=out_type, mesh=mesh,
                     scratch_types=scratch)


def _make_sc_deg():
    mesh = plsc.VectorSubcoreMesh(core_axis_name="c", subcore_axis_name="s")

    out_type = jax.ShapeDtypeStruct((NC, 8, N_PAD), jnp.float32)
    COLS = N_PAD // NS
    scratch = [
        pltpu.VMEM((E_W,), jnp.int32),
        pltpu.VMEM((N_PAD,), jnp.float32),
        pltpu.VMEM((NS * COLS,), jnp.float32),
        pltpu.VMEM((COLS,), jnp.float32),
        pltpu.VMEM_SHARED((NS * N_PAD,), jnp.float32),
    ]

    def body(dst, out_deg, slab_d, hist, red, obuf, deg_sh):
        cid = lax.axis_index("c")
        sid = lax.axis_index("s")
        wid = sid * NC + cid

        pltpu.sync_copy(dst.at[pl.ds(wid * E_W, E_W)], slab_d)

        zv = jnp.zeros((16,), jnp.float32)

        def zero_body(i, carry):
            hist[pl.ds(i * 16, 16)] = zv
            return carry

        lax.fori_loop(0, N_PAD // 16, zero_body, 0)

        onev = jnp.ones((16,), jnp.float32)

        def chunk_body(j, carry):
            for k in range(8):
                idxv = slab_d[pl.ds((j * 8 + k) * 16, 16)]
                plsc.addupdate_scatter(hist, [idxv], onev)
            return carry

        lax.fori_loop(0, E_W // CHUNK, chunk_body, 0)

        def rem_body(v, carry):
            idxv = slab_d[pl.ds((E_W // CHUNK) * CHUNK + v * 16, 16)]
            plsc.addupdate_scatter(hist, [idxv], onev)
            return carry

        lax.fori_loop(0, (E_W % CHUNK) // 16, rem_body, 0)

        pltpu.sync_copy(hist, deg_sh.at[pl.ds(sid * N_PAD, N_PAD)])
        plsc.subcore_barrier()

        c0 = sid * COLS

        def pull_body(j, carry):
            pltpu.sync_copy(deg_sh.at[pl.ds(j * N_PAD + c0, COLS)],
                            red.at[pl.ds(j * COLS, COLS)])
            return carry

        lax.fori_loop(0, NS, pull_body, 0)

        def sum_body(t, carry):
            s = red[pl.ds(t * 16, 16)]
            for j in range(1, NS):
                s = s + red[pl.ds(j * COLS + t * 16, 16)]
            obuf[pl.ds(t * 16, 16)] = s
            return carry

        lax.fori_loop(0, COLS // 16, sum_body, 0)

        pltpu.sync_copy(obuf, out_deg.at[cid, 0, pl.ds(c0, COLS)])

    return pl.kernel(body, out_type=out_type, mesh=mesh,
                     scratch_types=scratch,
                     compiler_params=pltpu.CompilerParams(
                         needs_layout_passes=False))


_sc_agg = _make_sc_agg()
_sc_deg = _make_sc_deg()


def _combine(feat, p, inv, w_self, w_neigh, b, relu):
    R = 2000

    def body(feat_ref, p_ref, inv_ref, ws_ref, wn_ref, b_ref, out_ref):
        neigh = (p_ref[0] + p_ref[1]) * inv_ref[...]
        acc = jnp.dot(feat_ref[...], ws_ref[...],
                      preferred_element_type=jnp.float32)
        acc += jnp.dot(neigh, wn_ref[...], preferred_element_type=jnp.float32)
        acc += b_ref[...]
        if relu:
            acc = jnp.maximum(acc, 0.0)
        out_ref[...] = acc

    return pl.pallas_call(
        body,
        grid=(N_NODES // R,),
        in_specs=[
            pl.BlockSpec((R, D), lambda i: (i, 0)),
            pl.BlockSpec((NC, R, D), lambda i: (0, i, 0)),
            pl.BlockSpec((R, 1), lambda i: (i, 0)),
            pl.BlockSpec((D, D), lambda i: (0, 0)),
            pl.BlockSpec((D, D), lambda i: (0, 0)),
            pl.BlockSpec((1, D), lambda i: (0, 0)),
        ],
        out_specs=pl.BlockSpec((R, D), lambda i: (i, 0)),
        out_shape=jax.ShapeDtypeStruct((N_NODES, D), jnp.float32),
    )(feat, p, inv, w_self, w_neigh, b.reshape(1, D))


@jax.jit
def _impl(x, src, dst, W1_self, W1_neigh, b1, W2_self, W2_neigh, b2):
    accp1 = _sc_agg(x, src, dst)
    degp = _sc_deg(dst)
    deg = degp[0, 0, :] + degp[1, 0, :]
    inv = (1.0 / jnp.maximum(deg, 1.0)).reshape(N_PAD, 1)
    h = _combine(x, accp1, inv, W1_self, W1_neigh, b1, relu=True)
    accp2 = _sc_agg(h, src, dst)
    return _combine(h, accp2, inv, W2_self, W2_neigh, b2, relu=False)


def kernel(x, edge_index, W1_self, W1_neigh, b1, W2_self, W2_neigh, b2):
    src = edge_index[0].astype(jnp.int32)
    dst = edge_index[1].astype(jnp.int32)
    return _impl(x, src, dst, W1_self, W1_neigh, b1,
                 W2_self, W2_neigh, b2)
